# baseline (speedup 1.0000x reference)
import functools

import jax
import jax.numpy as jnp
from jax import lax
from jax.experimental import pallas as pl
from jax.experimental.pallas import tpu as pltpu
from jax.experimental.pallas import tpu_sc as plsc

N = 10000
NP = 10240
D = 256
DH = 128
KITER = 10
E = 160000
NTILES = 16
RPT = NP // NTILES
CH = 64
NRB = 2560
NBLK = NRB // 16
PADN = N
NCHL = 184
LCAP = NCHL * CH
ACC = RPT * DH

_SC_MESH = dict(
    mesh=plsc.VectorSubcoreMesh(core_axis_name="c", subcore_axis_name="s"),
    compiler_params=pltpu.CompilerParams(
        needs_layout_passes=False, use_tc_tiling_on_sc=False),
)


def _rsqrt(x):
    i = plsc.bitcast(x, jnp.int32)
    i = jnp.int32(0x5F3759DF) - lax.shift_right_arithmetic(i, 1)
    y = plsc.bitcast(i, jnp.float32)
    for _ in range(3):
        y = y * (1.5 - 0.5 * x * y * y)
    return y


def _splat(ref, n):
    return plsc.load_gather(ref, [jnp.full((16,), n, jnp.int32)])


@functools.partial(
    pl.kernel,
    out_type=(
        jax.ShapeDtypeStruct((2 * NP, DH), jnp.float32),
        jax.ShapeDtypeStruct((2 * NP, DH), jnp.float32),
        jax.ShapeDtypeStruct((2 * NP * DH,), jnp.float32),
    ),
    scratch_types=[
        pltpu.VMEM_SHARED((NP,), jnp.float32),
        pltpu.VMEM_SHARED((NP,), jnp.float32),
        pltpu.VMEM((ACC,), jnp.float32),
        pltpu.VMEM((LCAP,), jnp.int32),
        pltpu.VMEM((LCAP,), jnp.int32),
        pltpu.VMEM((CH, DH), jnp.float32),
        pltpu.VMEM((CH, DH), jnp.float32),
        pltpu.VMEM((16, CH), jnp.int32),
        pltpu.VMEM((16, CH), jnp.int32),
        pltpu.VMEM((16 * DH,), jnp.float32),
        pltpu.VMEM((RPT,), jnp.float32),
        pltpu.VMEM((RPT,), jnp.float32),
        pltpu.VMEM((RPT,), jnp.float32),
        pltpu.VMEM((CH,), jnp.float32),
        pltpu.VMEM((CH,), jnp.int32),
        pltpu.SemaphoreType.DMA,
        pltpu.SemaphoreType.DMA,
    ],
    **_SC_MESH,
)
def _appnp_sc(feat_hbm, srcr, dstr, out_hbm, g_hbm, a2f,
              hist_s, hist_d, accum, src_l, dst_l, bufA, bufB,
              ringS, ringD, fAflat, hbuf, nsrc, ndst, ones, dbuf,
              semA, semB):
    c = lax.axis_index("c")
    s = lax.axis_index("s")
    nb = s * RPT
    gb = c * NP + nb
    iota16 = lax.iota(jnp.int32, 16)

    with jax.named_scope("hist"):
        @pl.loop(0, CH // 16)
        def _(i):
            ones[pl.ds(i * 16, 16)] = jnp.ones((16,), jnp.float32)

        @pl.loop(0, RPT // 16)
        def _(i):
            hbuf[pl.ds(i * 16, 16)] = jnp.zeros((16,), jnp.float32)

        pltpu.sync_copy(hbuf, hist_s.at[pl.ds(nb, RPT)])
        pltpu.sync_copy(hbuf, hist_d.at[pl.ds(nb, RPT)])
        plsc.subcore_barrier()

        @pl.loop(0, NBLK // NTILES)
        def _(bb):
            b = s * (NBLK // NTILES) + bb
            pltpu.sync_copy(srcr.at[pl.ds(b * 16, 16)], ringS)
            pltpu.sync_copy(dstr.at[pl.ds(b * 16, 16)], ringD)
            @pl.loop(0, 16)
            def _(r):
                pltpu.async_copy(ones, hist_s.at[ringS.at[r]], semA, add=True)
                pltpu.async_copy(ones, hist_d.at[ringD.at[r]], semB, add=True)

            @pl.loop(0, 16)
            def _(r):
                pltpu.make_async_copy(ones, hist_s.at[ringS.at[r]], semA).wait()
                pltpu.make_async_copy(ones, hist_d.at[ringD.at[r]], semB).wait()
        plsc.subcore_barrier()

    with jax.named_scope("norms"):
        pltpu.sync_copy(hist_s.at[pl.ds(nb, RPT)], hbuf)

        @pl.loop(0, RPT // 16)
        def _(i):
            cnt = hbuf[pl.ds(i * 16, 16)]
            nsrc[pl.ds(i * 16, 16)] = _rsqrt(jnp.maximum(cnt, 1.0))

        pltpu.sync_copy(hist_d.at[pl.ds(nb, RPT)], hbuf)

        @pl.loop(0, RPT // 16)
        def _(i):
            cnt = jnp.maximum(hbuf[pl.ds(i * 16, 16)], 1.0)
            ndst[pl.ds(i * 16, 16)] = _rsqrt(cnt)

    with jax.named_scope("g0a2"):
        @pl.loop(0, RPT // 16)
        def _(blk):
            pltpu.sync_copy(feat_hbm.at[pl.ds(gb + blk * 16, 16)],
                            bufA.at[pl.ds(0, 16)])

            @pl.loop(0, 16)
            def _(i):
                ws = _splat(nsrc, blk * 16 + i)
                wa = (1.0 / 9.0) / _splat(ndst, blk * 16 + i)
                for q in range(DH // 16):
                    v = bufA[i, pl.ds(q * 16, 16)]
                    bufB[i, pl.ds(q * 16, 16)] = v * ws
                    fAflat[pl.ds(i * DH + q * 16, 16)] = v * wa

            pltpu.sync_copy(bufB.at[pl.ds(0, 16)],
                            g_hbm.at[pl.ds(gb + blk * 16, 16)])
            pltpu.sync_copy(fAflat, a2f.at[pl.ds((gb + blk * 16) * DH, 16 * DH)])

    with jax.named_scope("bucket"):
        off = c * NP
        padv = off + PADN

        @pl.loop(0, LCAP // 16)
        def _(i):
            src_l[pl.ds(i * 16, 16)] = jnp.full((16,), padv, jnp.int32)
            dst_l[pl.ds(i * 16, 16)] = jnp.zeros((16,), jnp.int32)

        lo = s * RPT

        @pl.loop(0, NBLK, init_carry=jnp.int32(0))
        def _bucket(b, pos):
            pltpu.sync_copy(srcr.at[pl.ds(b * 16, 16)], ringS)
            pltpu.sync_copy(dstr.at[pl.ds(b * 16, 16)], ringD)

            @pl.loop(0, 16, init_carry=pos)
            def _rows(r, pos2):
                for g4 in range(CH // 16):
                    sv = ringS[r, pl.ds(g4 * 16, 16)]
                    dv = ringD[r, pl.ds(g4 * 16, 16)]
                    m = (dv >= lo) & (dv < lo + RPT)
                    keep = jnp.where(m, 1, 0).astype(jnp.int32)
                    cum = lax.cumsum(keep, axis=0)
                    tgt = pos2 + cum - 1
                    plsc.store_scatter(src_l, [tgt], sv + off, mask=m)
                    plsc.store_scatter(dst_l, [tgt], dv - lo, mask=m)
                    pos2 = pos2 + jnp.max(cum)
                return pos2

            return _rows

    cq = [q * 16 + iota16 for q in range(DH // 16)]

    def accumulate(buf, j):
        for q4 in range(CH // 16):
            dbuf[pl.ds(q4 * 16, 16)] = dst_l[pl.ds(j * CH + q4 * 16, 16)]

        @pl.loop(0, CH, unroll=4)
        def _(r):
            base = _splat(dbuf, r) * DH
            for q in range(DH // 16):
                plsc.addupdate_scatter(
                    accum, [base + cq[q]], buf[r, pl.ds(q * 16, 16)])

    def do_round(is_final):
        plsc.subcore_barrier()
        with jax.named_scope("seed"):
            pltpu.sync_copy(a2f.at[pl.ds(gb * DH, ACC)], accum)
        with jax.named_scope("sweep"):
            pltpu.async_copy(g_hbm.at[src_l.at[pl.ds(0, CH)]], bufA, semA)

            @pl.loop(0, NCHL, step=2)
            def _(j):
                pltpu.async_copy(
                    g_hbm.at[src_l.at[pl.ds((j + 1) * CH, CH)]], bufB, semB)
                pltpu.make_async_copy(
                    g_hbm.at[src_l.at[pl.ds(j * CH, CH)]], bufA, semA).wait()
                accumulate(bufA, j)

                @pl.when(j + 2 < NCHL)
                def _():
                    pltpu.async_copy(
                        g_hbm.at[src_l.at[pl.ds((j + 2) * CH, CH)]], bufA, semA)

                pltpu.make_async_copy(
                    g_hbm.at[src_l.at[pl.ds((j + 1) * CH, CH)]], bufB, semB).wait()
                accumulate(bufB, j + 1)

        dst_arr = out_hbm if is_final else g_hbm
        with jax.named_scope("finish"):
            @pl.loop(0, RPT // CH)
            def _(blk):
                @pl.loop(0, CH)
                def _(i):
                    nl = blk * CH + i
                    w = _splat(ndst, nl) * 0.9
                    if not is_final:
                        w = w * _splat(nsrc, nl)
                    for q in range(DH // 16):
                        bufB[i, pl.ds(q * 16, 16)] = (
                            accum[pl.ds(nl * DH + q * 16, 16)] * w)

                pltpu.sync_copy(bufB, dst_arr.at[pl.ds(gb + blk * CH, CH)])

    @pl.loop(0, KITER - 1)
    def _(k):
        do_round(False)

    do_round(True)


def kernel(feat, edge_index):
    feat = feat.astype(jnp.float32)
    f2 = feat.reshape(N, 2, DH).transpose(1, 0, 2)
    f2 = jnp.pad(f2, ((0, 0), (0, NP - N), (0, 0)))
    feat_flat = f2.reshape(2 * NP, DH)
    src = edge_index[0].astype(jnp.int32)
    dst = edge_index[1].astype(jnp.int32)
    srcr = jnp.pad(src, (0, NRB * CH - E), constant_values=PADN).reshape(NRB, CH)
    dstr = jnp.pad(dst, (0, NRB * CH - E), constant_values=PADN).reshape(NRB, CH)

    out, _, _ = _appnp_sc(feat_flat, srcr, dstr)
    return jnp.concatenate([out[0:N], out[NP:NP + N]], axis=1)


# speedup vs baseline: 1.0438x; 1.0438x over previous
import functools

import jax
import jax.numpy as jnp
from jax import lax
from jax.experimental import pallas as pl
from jax.experimental.pallas import tpu as pltpu
from jax.experimental.pallas import tpu_sc as plsc

N = 10000
NP = 10240
D = 256
DH = 128
KITER = 10
E = 160000
NTILES = 16
RPT = NP // NTILES
CH = 64
NRB = 2560
NBLK = NRB // 16
PADN = N
NCHL = 184
LCAP = NCHL * CH
ACC = RPT * DH

_SC_MESH = dict(
    mesh=plsc.VectorSubcoreMesh(core_axis_name="c", subcore_axis_name="s"),
    compiler_params=pltpu.CompilerParams(
        needs_layout_passes=False, use_tc_tiling_on_sc=False),
)


def _rsqrt(x):
    i = plsc.bitcast(x, jnp.int32)
    i = jnp.int32(0x5F3759DF) - lax.shift_right_arithmetic(i, 1)
    y = plsc.bitcast(i, jnp.float32)
    for _ in range(3):
        y = y * (1.5 - 0.5 * x * y * y)
    return y


def _splat(ref, n):
    return plsc.load_gather(ref, [jnp.full((16,), n, jnp.int32)])


@functools.partial(
    pl.kernel,
    out_type=(
        jax.ShapeDtypeStruct((2 * NP, DH), jnp.float32),
        jax.ShapeDtypeStruct((2 * NP, DH), jnp.float32),
        jax.ShapeDtypeStruct((2 * NP * DH,), jnp.float32),
    ),
    scratch_types=[
        pltpu.VMEM_SHARED((NP,), jnp.float32),
        pltpu.VMEM_SHARED((NP,), jnp.float32),
        pltpu.VMEM((ACC,), jnp.float32),
        pltpu.VMEM((LCAP,), jnp.int32),
        pltpu.VMEM((LCAP,), jnp.int32),
        pltpu.VMEM((CH, DH), jnp.float32),
        pltpu.VMEM((CH, DH), jnp.float32),
        pltpu.VMEM((16, CH), jnp.int32),
        pltpu.VMEM((16, CH), jnp.int32),
        pltpu.VMEM((16 * DH,), jnp.float32),
        pltpu.VMEM((RPT,), jnp.float32),
        pltpu.VMEM((RPT,), jnp.float32),
        pltpu.VMEM((RPT,), jnp.float32),
        pltpu.VMEM((CH,), jnp.float32),
        pltpu.VMEM((CH,), jnp.int32),
        pltpu.SemaphoreType.DMA,
        pltpu.SemaphoreType.DMA,
    ],
    **_SC_MESH,
)
def _appnp_sc(feat_hbm, srcr, dstr, out_hbm, g_hbm, a2f,
              hist_s, hist_d, accum, src_l, dst_l, bufA, bufB,
              ringS, ringD, fAflat, hbuf, nsrc, ndst, ones, dbuf,
              semA, semB):
    c = lax.axis_index("c")
    s = lax.axis_index("s")
    nb = s * RPT
    gb = c * NP + nb
    iota16 = lax.iota(jnp.int32, 16)

    with jax.named_scope("hist"):
        @pl.loop(0, CH // 16)
        def _(i):
            ones[pl.ds(i * 16, 16)] = jnp.ones((16,), jnp.float32)

        @pl.loop(0, RPT // 16)
        def _(i):
            hbuf[pl.ds(i * 16, 16)] = jnp.zeros((16,), jnp.float32)

        pltpu.sync_copy(hbuf, hist_s.at[pl.ds(nb, RPT)])
        pltpu.sync_copy(hbuf, hist_d.at[pl.ds(nb, RPT)])
        plsc.subcore_barrier()

        @pl.loop(0, NBLK // NTILES)
        def _(bb):
            b = s * (NBLK // NTILES) + bb
            pltpu.sync_copy(srcr.at[pl.ds(b * 16, 16)], ringS)
            pltpu.sync_copy(dstr.at[pl.ds(b * 16, 16)], ringD)
            @pl.loop(0, 16)
            def _(r):
                pltpu.async_copy(ones, hist_s.at[ringS.at[r]], semA, add=True)
                pltpu.async_copy(ones, hist_d.at[ringD.at[r]], semB, add=True)

            @pl.loop(0, 16)
            def _(r):
                pltpu.make_async_copy(ones, hist_s.at[ringS.at[r]], semA).wait()
                pltpu.make_async_copy(ones, hist_d.at[ringD.at[r]], semB).wait()
        plsc.subcore_barrier()

    with jax.named_scope("norms"):
        pltpu.sync_copy(hist_s.at[pl.ds(nb, RPT)], hbuf)

        @pl.loop(0, RPT // 16)
        def _(i):
            cnt = hbuf[pl.ds(i * 16, 16)]
            nsrc[pl.ds(i * 16, 16)] = _rsqrt(jnp.maximum(cnt, 1.0))

        pltpu.sync_copy(hist_d.at[pl.ds(nb, RPT)], hbuf)

        @pl.loop(0, RPT // 16)
        def _(i):
            cnt = jnp.maximum(hbuf[pl.ds(i * 16, 16)], 1.0)
            ndst[pl.ds(i * 16, 16)] = _rsqrt(cnt)

    with jax.named_scope("g0a2"):
        @pl.loop(0, RPT // 16)
        def _(blk):
            pltpu.sync_copy(feat_hbm.at[pl.ds(gb + blk * 16, 16)],
                            bufA.at[pl.ds(0, 16)])

            @pl.loop(0, 16)
            def _(i):
                ws = _splat(nsrc, blk * 16 + i)
                wa = (1.0 / 9.0) / _splat(ndst, blk * 16 + i)
                for q in range(DH // 16):
                    v = bufA[i, pl.ds(q * 16, 16)]
                    bufB[i, pl.ds(q * 16, 16)] = v * ws
                    fAflat[pl.ds(i * DH + q * 16, 16)] = v * wa

            pltpu.sync_copy(bufB.at[pl.ds(0, 16)],
                            g_hbm.at[pl.ds(gb + blk * 16, 16)])
            pltpu.sync_copy(fAflat, a2f.at[pl.ds((gb + blk * 16) * DH, 16 * DH)])

    with jax.named_scope("bucket"):
        off = c * NP
        padv = off + PADN

        @pl.loop(0, LCAP // 16)
        def _(i):
            src_l[pl.ds(i * 16, 16)] = jnp.full((16,), padv, jnp.int32)
            dst_l[pl.ds(i * 16, 16)] = jnp.zeros((16,), jnp.int32)

        lo = s * RPT

        @pl.loop(0, NBLK, init_carry=jnp.int32(0))
        def _bucket(b, pos):
            pltpu.sync_copy(srcr.at[pl.ds(b * 16, 16)], ringS)
            pltpu.sync_copy(dstr.at[pl.ds(b * 16, 16)], ringD)

            @pl.loop(0, 16, init_carry=pos)
            def _rows(r, pos2):
                for g4 in range(CH // 16):
                    sv = ringS[r, pl.ds(g4 * 16, 16)]
                    dv = ringD[r, pl.ds(g4 * 16, 16)]
                    m = (dv >= lo) & (dv < lo + RPT)
                    keep = jnp.where(m, 1, 0).astype(jnp.int32)
                    cum = lax.cumsum(keep, axis=0)
                    tgt = pos2 + cum - 1
                    plsc.store_scatter(src_l, [tgt], sv + off, mask=m)
                    plsc.store_scatter(dst_l, [tgt], dv - lo, mask=m)
                    pos2 = pos2 + jnp.max(cum)
                return pos2

            return _rows

    cq = [q * 16 + iota16 for q in range(DH // 16)]
    lanes = [jnp.full((16,), r16, jnp.int32) for r16 in range(16)]

    def accumulate(buf, j):
        @pl.loop(0, CH // 16)
        def _(g4):
            bases = dst_l[pl.ds(j * CH + g4 * 16, 16)] * DH
            for r16 in range(16):
                base = lax.gather(
                    bases, lanes[r16][:, None],
                    lax.GatherDimensionNumbers(
                        offset_dims=(), collapsed_slice_dims=(0,),
                        start_index_map=(0,)),
                    (1,), mode=lax.GatherScatterMode.PROMISE_IN_BOUNDS)
                r = g4 * 16 + r16
                vs = [buf[r, pl.ds(q * 16, 16)] for q in range(DH // 16)]
                for q in range(DH // 16):
                    plsc.addupdate_scatter(accum, [base + cq[q]], vs[q])

    def do_round(is_final):
        plsc.subcore_barrier()
        with jax.named_scope("seed"):
            pltpu.sync_copy(a2f.at[pl.ds(gb * DH, ACC)], accum)
        with jax.named_scope("sweep"):
            pltpu.async_copy(g_hbm.at[src_l.at[pl.ds(0, CH)]], bufA, semA)

            @pl.loop(0, NCHL, step=2)
            def _(j):
                pltpu.async_copy(
                    g_hbm.at[src_l.at[pl.ds((j + 1) * CH, CH)]], bufB, semB)
                pltpu.make_async_copy(
                    g_hbm.at[src_l.at[pl.ds(j * CH, CH)]], bufA, semA).wait()
                accumulate(bufA, j)

                @pl.when(j + 2 < NCHL)
                def _():
                    pltpu.async_copy(
                        g_hbm.at[src_l.at[pl.ds((j + 2) * CH, CH)]], bufA, semA)

                pltpu.make_async_copy(
                    g_hbm.at[src_l.at[pl.ds((j + 1) * CH, CH)]], bufB, semB).wait()
                accumulate(bufB, j + 1)

        dst_arr = out_hbm if is_final else g_hbm
        with jax.named_scope("finish"):
            @pl.loop(0, RPT // CH)
            def _(blk):
                @pl.loop(0, CH)
                def _(i):
                    nl = blk * CH + i
                    w = _splat(ndst, nl) * 0.9
                    if not is_final:
                        w = w * _splat(nsrc, nl)
                    for q in range(DH // 16):
                        bufB[i, pl.ds(q * 16, 16)] = (
                            accum[pl.ds(nl * DH + q * 16, 16)] * w)

                pltpu.sync_copy(bufB, dst_arr.at[pl.ds(gb + blk * CH, CH)])

    @pl.loop(0, KITER - 1)
    def _(k):
        do_round(False)

    do_round(True)


def kernel(feat, edge_index):
    feat = feat.astype(jnp.float32)
    f2 = feat.reshape(N, 2, DH).transpose(1, 0, 2)
    f2 = jnp.pad(f2, ((0, 0), (0, NP - N), (0, 0)))
    feat_flat = f2.reshape(2 * NP, DH)
    src = edge_index[0].astype(jnp.int32)
    dst = edge_index[1].astype(jnp.int32)
    srcr = jnp.pad(src, (0, NRB * CH - E), constant_values=PADN).reshape(NRB, CH)
    dstr = jnp.pad(dst, (0, NRB * CH - E), constant_values=PADN).reshape(NRB, CH)

    out, _, _ = _appnp_sc(feat_flat, srcr, dstr)
    return jnp.concatenate([out[0:N], out[NP:NP + N]], axis=1)
